# baseline (speedup 1.0000x reference)
import functools

import jax
import jax.numpy as jnp
from jax import lax
from jax.experimental import pallas as pl
from jax.experimental.pallas import tpu as pltpu
from jax.experimental.pallas import tpu_sc as plsc

BM = 256
E = 16


def _sc_gather_rows(src, idx):
    M = idx.shape[0]
    D = src.shape[1]
    info = plsc.get_sparse_core_info()
    NC, NS = info.num_cores, info.num_subcores
    NW = NC * NS
    per_w = M // NW
    ch = per_w
    while ch * D * 4 > 256 * 1024:
        ch //= 2
    mesh = plsc.VectorSubcoreMesh(core_axis_name="c", subcore_axis_name="s")

    @functools.partial(
        pl.kernel,
        out_type=jax.ShapeDtypeStruct((M, D), src.dtype),
        mesh=mesh,
        scratch_types=[
            pltpu.VMEM((ch,), jnp.int32),
            pltpu.VMEM((ch, D), src.dtype),
            pltpu.SemaphoreType.DMA,
        ],
    )
    def body(src_hbm, idx_hbm, out_hbm, idx_v, rows_v, sem):
        wid = lax.axis_index("s") * NC + lax.axis_index("c")
        for c in range(per_w // ch):
            base = wid * per_w + c * ch
            pltpu.sync_copy(idx_hbm.at[pl.ds(base, ch)], idx_v)
            pltpu.async_copy(src_hbm.at[idx_v], rows_v, sem).wait()
            pltpu.sync_copy(rows_v, out_hbm.at[pl.ds(base, ch)])

    return body(src, idx)


def _sc_scatter_out(h0, h1, inv):
    Mh, D = h0.shape
    M = 2 * Mh
    info = plsc.get_sparse_core_info()
    NC, NS = info.num_cores, info.num_subcores
    NW = NC * NS
    per_w = Mh // NW
    inv2 = inv.reshape(M // per_w, per_w)
    mesh = plsc.VectorSubcoreMesh(core_axis_name="c", subcore_axis_name="s")

    @functools.partial(
        pl.kernel,
        out_type=jax.ShapeDtypeStruct((M, D), h0.dtype),
        mesh=mesh,
        scratch_types=[
            pltpu.VMEM((per_w,), jnp.int32),
            pltpu.VMEM((per_w, D), h0.dtype),
            pltpu.SemaphoreType.DMA,
        ],
    )
    def body(h0_hbm, h1_hbm, idx_hbm, out_hbm, idx_v, rows_v, sem):
        wid = lax.axis_index("s") * NC + lax.axis_index("c")
        for p, src in enumerate((h0_hbm, h1_hbm)):
            pltpu.sync_copy(idx_hbm.at[p * NW + wid], idx_v)
            pltpu.sync_copy(src.at[pl.ds(wid * per_w, per_w)], rows_v)
            pltpu.async_copy(rows_v, out_hbm.at[idx_v], sem).wait()

    return body(h0, h1, inv2)


def _gmm_body(grp_ref, tile_ref, lo_ref, hi_ref, x_ref, w_ref, o_ref):
    i = pl.program_id(0)
    rows = lax.broadcasted_iota(jnp.int32, (BM, 1), 0)
    mask = (rows >= lo_ref[i]) & (rows < hi_ref[i])
    p = jnp.dot(x_ref[...], w_ref[0], preferred_element_type=jnp.float32)
    o_ref[...] = jnp.where(mask, p, o_ref[...])


def _tc_gmm(x_sorted, w, meta):
    grp, tile, lo, hi, steps = meta
    M, K = x_sorted.shape
    _, _, N = w.shape
    grid_spec = pltpu.PrefetchScalarGridSpec(
        num_scalar_prefetch=4,
        grid=(steps,),
        in_specs=[
            pl.BlockSpec((BM, K), lambda i, grp, tile, lo, hi: (tile[i], 0)),
            pl.BlockSpec((1, K, N), lambda i, grp, tile, lo, hi: (grp[i], 0, 0)),
        ],
        out_specs=pl.BlockSpec((BM, N), lambda i, grp, tile, lo, hi: (tile[i], 0)),
    )
    return pl.pallas_call(
        _gmm_body,
        grid_spec=grid_spec,
        out_shape=jax.ShapeDtypeStruct((M, N), jnp.float32),
        compiler_params=pltpu.CompilerParams(
            dimension_semantics=("arbitrary",),
        ),
    )(grp, tile, lo, hi, x_sorted, w)


def _token_positions(sel, M):
    i32 = jnp.int32
    B = 512
    G = M // B
    oh = (sel[:, None] == jnp.arange(E, dtype=sel.dtype)[None, :]).astype(jnp.float32)
    ohb = oh.reshape(G, B, E)
    tri = jnp.tril(jnp.ones((B, B), jnp.float32))
    within = lax.dot_general(tri, ohb, (((1,), (1,)), ((), ())))
    within = within.transpose(1, 0, 2)
    blocksum = within[:, -1, :]
    blockpref = jnp.cumsum(blocksum, axis=0) - blocksum
    cumf = (within + blockpref[:, None, :]).reshape(M, E)
    cnt = (blocksum[-1] + blockpref[-1]).astype(i32)
    starts = jnp.concatenate([jnp.zeros(1, i32), jnp.cumsum(cnt)[:-1].astype(i32)])
    ends = starts + cnt
    rank = jnp.sum(cumf * oh, axis=1).astype(i32) - 1
    pos = jnp.sum(starts[None, :].astype(jnp.float32) * oh, axis=1).astype(i32) + rank
    return pos.astype(i32), starts, ends


def _staircase(starts, ends, A, Mh):
    i32 = jnp.int32
    T = Mh // BM
    S = T + E - 1
    cs = jnp.clip(starts - A, 0, Mh)
    ce = jnp.clip(ends - A, 0, Mh)
    first_tile = cs // BM
    ntiles = jnp.where(ce > cs, (ce + BM - 1) // BM - first_tile, 0)
    incl = jnp.cumsum(ntiles)
    total = incl[-1]
    step_start = incl - ntiles
    i = jnp.arange(S, dtype=i32)
    g = jnp.minimum(jnp.sum(incl[None, :] <= i[:, None], axis=1), E - 1)
    tile = first_tile[g] + (i - step_start[g])
    valid = i < total
    tile = jnp.where(valid, tile, T - 1).astype(i32)
    lo = jnp.where(valid, jnp.clip(cs[g] - tile * BM, 0, BM), 0).astype(i32)
    hi = jnp.where(valid, jnp.clip(ce[g] - tile * BM, 0, BM), 0).astype(i32)
    grp = jnp.where(valid, g, E - 1).astype(i32)
    return grp, tile, lo, hi, S


def kernel(x, sel, w):
    M, K = x.shape
    pos, starts, ends = _token_positions(sel, M)
    inv = jnp.zeros((M,), jnp.int32).at[pos].set(jnp.arange(M, dtype=jnp.int32))
    half = M // 2
    meta0 = _staircase(starts, ends, 0, half)
    meta1 = _staircase(starts, ends, half, half)
    xs0 = _sc_gather_rows(x, inv[:half])
    o0 = _tc_gmm(xs0, w, meta0)
    xs1 = _sc_gather_rows(x, inv[half:])
    o1 = _tc_gmm(xs1, w, meta1)
    return _sc_scatter_out(o0, o1, inv)


# speedup vs baseline: 1.3651x; 1.3651x over previous
import functools

import jax
import jax.numpy as jnp
from jax import lax
from jax.experimental import pallas as pl
from jax.experimental.pallas import tpu as pltpu
from jax.experimental.pallas import tpu_sc as plsc

BM = 256


def _sc_scatter_rows(src, idx):
    M, D = src.shape
    info = plsc.get_sparse_core_info()
    NC, NS = info.num_cores, info.num_subcores
    NW = NC * NS
    per_w = M // NW
    ch = per_w
    while 2 * ch * D * 4 > 500 * 1024:
        ch //= 2
    nch = per_w // ch
    idx2 = idx.reshape(M // ch, ch)
    mesh = plsc.VectorSubcoreMesh(core_axis_name="c", subcore_axis_name="s")

    @functools.partial(
        pl.kernel,
        out_type=jax.ShapeDtypeStruct((M, D), src.dtype),
        mesh=mesh,
        scratch_types=[
            pltpu.VMEM((nch, ch), jnp.int32),
            pltpu.VMEM((ch, D), src.dtype),
            pltpu.VMEM((ch, D), src.dtype),
            pltpu.SemaphoreType.DMA,
            pltpu.SemaphoreType.DMA,
            pltpu.SemaphoreType.DMA,
            pltpu.SemaphoreType.DMA,
        ],
    )
    def body(src_hbm, idx_hbm, out_hbm, idx_v, row0, row1, ls0, ls1, ss0, ss1):
        wid = lax.axis_index("s") * NC + lax.axis_index("c")
        base = wid * per_w
        pltpu.sync_copy(idx_hbm.at[pl.ds(wid * nch, nch)], idx_v)
        rows = (row0, row1)
        lsem = (ls0, ls1)
        ssem = (ss0, ss1)
        loads = [None, None]
        scats = [None, None]
        loads[0] = pltpu.async_copy(src_hbm.at[pl.ds(base, ch)], rows[0], lsem[0])
        for c in range(nch):
            b = c & 1
            nb = 1 - b
            if c + 1 < nch:
                if scats[nb] is not None:
                    scats[nb].wait()
                loads[nb] = pltpu.async_copy(
                    src_hbm.at[pl.ds(base + (c + 1) * ch, ch)], rows[nb], lsem[nb])
            loads[b].wait()
            scats[b] = pltpu.async_copy(rows[b], out_hbm.at[idx_v.at[c]], ssem[b])
        for s in scats:
            if s is not None:
                s.wait()

    return body(src, idx2)


def _sc_gather_rows(src, idx):
    M = idx.shape[0]
    D = src.shape[1]
    info = plsc.get_sparse_core_info()
    NC, NS = info.num_cores, info.num_subcores
    NW = NC * NS
    per_w = M // NW
    ch = per_w
    while ch * D * 4 > 256 * 1024:
        ch //= 2
    mesh = plsc.VectorSubcoreMesh(core_axis_name="c", subcore_axis_name="s")

    @functools.partial(
        pl.kernel,
        out_type=jax.ShapeDtypeStruct((M, D), src.dtype),
        mesh=mesh,
        scratch_types=[
            pltpu.VMEM((ch,), jnp.int32),
            pltpu.VMEM((ch, D), src.dtype),
            pltpu.SemaphoreType.DMA,
        ],
    )
    def body(src_hbm, idx_hbm, out_hbm, idx_v, rows_v, sem):
        wid = lax.axis_index("s") * NC + lax.axis_index("c")
        for c in range(per_w // ch):
            base = wid * per_w + c * ch
            pltpu.sync_copy(idx_hbm.at[pl.ds(base, ch)], idx_v)
            pltpu.async_copy(src_hbm.at[idx_v], rows_v, sem).wait()
            pltpu.sync_copy(rows_v, out_hbm.at[pl.ds(base, ch)])

    return body(src, idx)


def _gmm_body(grp_ref, tile_ref, lo_ref, hi_ref, x_ref, w_ref, o_ref):
    i = pl.program_id(0)
    rows = lax.broadcasted_iota(jnp.int32, (BM, 1), 0)
    mask = (rows >= lo_ref[i]) & (rows < hi_ref[i])
    p = jnp.dot(x_ref[...], w_ref[0], preferred_element_type=jnp.float32)
    o_ref[...] = jnp.where(mask, p, o_ref[...])


def _tc_gmm(x_sorted, w, grp, tile, lo, hi, steps):
    M, K = x_sorted.shape
    E, _, N = w.shape
    grid_spec = pltpu.PrefetchScalarGridSpec(
        num_scalar_prefetch=4,
        grid=(steps,),
        in_specs=[
            pl.BlockSpec((BM, K), lambda i, grp, tile, lo, hi: (tile[i], 0)),
            pl.BlockSpec((1, K, N), lambda i, grp, tile, lo, hi: (grp[i], 0, 0)),
        ],
        out_specs=pl.BlockSpec((BM, N), lambda i, grp, tile, lo, hi: (tile[i], 0)),
    )
    return pl.pallas_call(
        _gmm_body,
        grid_spec=grid_spec,
        out_shape=jax.ShapeDtypeStruct((M, N), jnp.float32),
        compiler_params=pltpu.CompilerParams(
            dimension_semantics=("arbitrary",),
        ),
    )(grp, tile, lo, hi, x_sorted, w)


def _routing_metadata(sel, M, E):
    i32 = jnp.int32
    B = 512
    G = M // B
    oh = (sel[:, None] == jnp.arange(E, dtype=sel.dtype)[None, :]).astype(jnp.float32)
    ohb = oh.reshape(G, B, E)
    tri = jnp.tril(jnp.ones((B, B), jnp.float32))
    within = jax.lax.dot_general(tri, ohb, (((1,), (1,)), ((), ())))
    within = within.transpose(1, 0, 2)
    blocksum = within[:, -1, :]
    blockpref = jnp.cumsum(blocksum, axis=0) - blocksum
    cumf = (within + blockpref[:, None, :]).reshape(M, E)
    cnt = (blocksum[-1] + blockpref[-1]).astype(i32)
    starts = jnp.concatenate([jnp.zeros(1, i32), jnp.cumsum(cnt)[:-1].astype(i32)])
    ends = starts + cnt
    rank = jnp.sum(cumf * oh, axis=1).astype(i32) - 1
    pos = jnp.sum(starts[None, :].astype(jnp.float32) * oh, axis=1).astype(i32) + rank

    T = M // BM
    S = T + E - 1
    first_tile = starts // BM
    ntiles = jnp.where(cnt > 0, (ends + BM - 1) // BM - first_tile, 0)
    incl = jnp.cumsum(ntiles)
    total = incl[-1]
    step_start = incl - ntiles
    i = jnp.arange(S, dtype=i32)
    g = jnp.minimum(jnp.sum(incl[None, :] <= i[:, None], axis=1), E - 1)
    tile = first_tile[g] + (i - step_start[g])
    valid = i < total
    tile = jnp.where(valid, tile, T - 1).astype(i32)
    lo = jnp.where(valid, jnp.clip(starts[g] - tile * BM, 0, BM), 0).astype(i32)
    hi = jnp.where(valid, jnp.clip(ends[g] - tile * BM, 0, BM), 0).astype(i32)
    grp = jnp.where(valid, g, E - 1).astype(i32)
    return pos.astype(i32), grp, tile, lo, hi, S


def kernel(x, sel, w):
    M, K = x.shape
    E, _, N = w.shape
    pos, grp, tile, lo, hi, steps = _routing_metadata(sel, M, E)
    x_sorted = _sc_scatter_rows(x, pos)
    out_sorted = _tc_gmm(x_sorted, w, grp, tile, lo, hi, steps)
    return _sc_gather_rows(out_sorted, pos)
